# baseline (speedup 1.0000x reference)
import jax
import jax.numpy as jnp
from jax import lax
from jax.experimental import pallas as pl
from jax.experimental.pallas import tpu as pltpu
from jax.experimental.pallas import tpu_sc as plsc

NC = 2
NS = 16
NW = NC * NS
L = 16
K = 128
SC0_FRAC = 0.63


def _sc_mesh():
    return plsc.VectorSubcoreMesh(core_axis_name="c", subcore_axis_name="s",
                                  num_cores=NC, num_subcores=NS)


def _make_deg_kernel(n_pad, q0, q1, q_max):
    rpt = n_pad // NS

    def body(dst_hbm, out_hbm, dst_v, ones_v, zero_v, deg_sp):
        c = lax.axis_index("c")
        s = lax.axis_index("s")
        wid = c * NS + s
        qw = jnp.where(c == 0, q0, q1)
        pltpu.sync_copy(dst_hbm.at[wid], dst_v)

        def init_ones(i, carry):
            ones_v[pl.ds(i * L, L)] = jnp.ones((L,), jnp.float32)
            return carry

        lax.fori_loop(0, K // L, init_ones, 0)

        def init_zero(i, carry):
            zero_v[pl.ds(i * L, L)] = jnp.zeros((L,), jnp.float32)
            return carry

        lax.fori_loop(0, rpt // L, init_zero, 0)
        pltpu.sync_copy(zero_v, deg_sp.at[pl.ds(s * rpt, rpt)])
        plsc.subcore_barrier()

        def chunk(j, carry):
            pltpu.sync_copy(ones_v, deg_sp.at[dst_v.at[j]], add=True)
            return carry

        lax.fori_loop(0, qw, chunk, 0)
        plsc.subcore_barrier()
        pltpu.sync_copy(deg_sp.at[pl.ds(s * rpt, rpt)],
                        out_hbm.at[c, pl.ds(s * rpt, rpt)])

    return pl.kernel(
        body,
        out_type=jax.ShapeDtypeStruct((NC, n_pad), jnp.float32),
        mesh=_sc_mesh(),
        scratch_types=[
            pltpu.VMEM((q_max, K), jnp.int32),
            pltpu.VMEM((K,), jnp.float32),
            pltpu.VMEM((rpt,), jnp.float32),
            pltpu.VMEM_SHARED((n_pad,), jnp.float32),
        ],
    )


def _make_scatter_kernel(n_pad, d, q0, q1, q_half):
    rpt = n_pad // NS

    def body(zt_hbm, src_hbm, dst_hbm, out_hbm, src_v, dst_v, rows0, rows1,
             acc_sp, sem0, sem1):
        c = lax.axis_index("c")
        s = lax.axis_index("s")
        wid = c * NS + s
        qw = jnp.where(c == 0, q0, q1)
        pltpu.sync_copy(zt_hbm.at[pl.ds(s * rpt, rpt)],
                        acc_sp.at[pl.ds(s * rpt, rpt)])
        plsc.subcore_barrier()

        def do_half(base, b):
            pltpu.sync_copy(src_hbm.at[wid, pl.ds(base, q_half)], src_v)
            pltpu.sync_copy(dst_hbm.at[wid, pl.ds(base, q_half)], dst_v)

            @pl.when(b > 0)
            def _():
                pltpu.async_copy(zt_hbm.at[src_v.at[0]], rows0, sem0)

            @pl.when(b > 1)
            def _():
                pltpu.async_copy(zt_hbm.at[src_v.at[1]], rows1, sem1)

            def chunk(j, carry):
                def step(rows, sem):
                    pltpu.make_async_copy(zt_hbm.at[src_v.at[j]], rows,
                                          sem).wait()
                    pltpu.sync_copy(rows, acc_sp.at[dst_v.at[j]], add=True)

                    @pl.when(j + 2 < b)
                    def _():
                        pltpu.async_copy(zt_hbm.at[src_v.at[j + 2]], rows,
                                         sem)

                @pl.when(j % 2 == 0)
                def _():
                    step(rows0, sem0)

                @pl.when(j % 2 == 1)
                def _():
                    step(rows1, sem1)

                return carry

            lax.fori_loop(0, b, chunk, 0)

        b0 = jnp.minimum(qw, q_half)
        do_half(0, b0)
        do_half(q_half, qw - b0)
        plsc.subcore_barrier()
        pltpu.sync_copy(acc_sp.at[pl.ds(s * rpt, rpt)],
                        out_hbm.at[c, pl.ds(s * rpt, rpt)])

    return pl.kernel(
        body,
        out_type=jax.ShapeDtypeStruct((NC, n_pad, d), jnp.float32),
        mesh=_sc_mesh(),
        scratch_types=[
            pltpu.VMEM((q_half, K), jnp.int32),
            pltpu.VMEM((q_half, K), jnp.int32),
            pltpu.VMEM((K, d), jnp.float32),
            pltpu.VMEM((K, d), jnp.float32),
            pltpu.VMEM_SHARED((n_pad, d), jnp.float32),
            pltpu.SemaphoreType.DMA,
            pltpu.SemaphoreType.DMA,
        ],
    )


def _make_pre_kernel(n, n_pad, h):
    def body(p0, p1, x, w, dis_out, zt_out):
        deg = p0[...] + p1[...] + 1.0
        rid = lax.broadcasted_iota(jnp.int32, (n_pad, 1), 0)
        dis = jnp.where(rid < n, lax.rsqrt(jnp.maximum(deg, 1e-12)), 0.0)
        dis_out[...] = dis
        zt_out[...] = jnp.dot(x[...], w[...],
                              preferred_element_type=jnp.float32) * dis

    return pl.pallas_call(
        body,
        out_shape=[
            jax.ShapeDtypeStruct((n_pad, 1), jnp.float32),
            jax.ShapeDtypeStruct((n_pad, h), jnp.float32),
        ],
    )


def _make_mid_kernel(n, n_pad, h, scale_out):
    def body(a0, a1, ztp, dis, b, g, be, w, bout, out):
        agg = a0[...] + a1[...] - ztp[...]
        y = agg * dis[...] + b[...]
        r = jnp.maximum(y, 0.0)
        mask = (lax.broadcasted_iota(jnp.int32, (n_pad, 1), 0)
                < n).astype(jnp.float32)
        mu = jnp.sum(r * mask, axis=0, keepdims=True) * (1.0 / n)
        dvn = (r - mu) * mask
        var = jnp.sum(dvn * dvn, axis=0, keepdims=True) * (1.0 / n)
        bn = (r - mu) * lax.rsqrt(var + 1e-5) * g[...] + be[...]
        res = jnp.dot(bn, w[...], preferred_element_type=jnp.float32)
        res = res + bout[...]
        if scale_out:
            res = res * dis[...]
        out[...] = res

    return pl.pallas_call(
        body,
        out_shape=jax.ShapeDtypeStruct((n_pad, h), jnp.float32),
    )


def kernel(x, edge_index, W1, b1, g1, be1, W2, b2, g2, be2, W3, b3, g3, be3,
           Wc, bc):
    n, d = x.shape
    h = W1.shape[1]
    e = edge_index.shape[1]
    c_out = Wc.shape[1]

    n_pad = ((n + 1 + 255) // 256) * 256
    nch = (e + NS * K - 1) // (NS * K)
    q0 = max(1, min(nch - 1, round(nch * SC0_FRAC)))
    q1 = nch - q0
    q_max = max(q0, q1)
    q_half = (((q_max + 1) // 2 + 7) // 8) * 8
    q_pad = 2 * q_half
    e_cov = NS * nch * K

    src = edge_index[0]
    dst = edge_index[1]
    pad_idx = jnp.full((e_cov - e,), n, jnp.int32)

    def slabs(idx):
        flat = jnp.concatenate([idx, pad_idx])
        c0 = flat[:NS * q0 * K].reshape(NS, q0, K)
        c1 = flat[NS * q0 * K:].reshape(NS, q1, K)
        pad0 = jnp.full((NS, q_pad - q0, K), n, jnp.int32)
        pad1 = jnp.full((NS, q_pad - q1, K), n, jnp.int32)
        return jnp.concatenate([jnp.concatenate([c0, pad0], axis=1),
                                jnp.concatenate([c1, pad1], axis=1)], axis=0)

    src_p = slabs(src)
    dst_p = slabs(dst)
    x_p = jnp.zeros((n_pad, d), jnp.float32).at[:n].set(x)
    wc_p = jnp.zeros((h, h), jnp.float32).at[:, :c_out].set(Wc)
    bc_p = jnp.zeros((1, h), jnp.float32).at[0, :c_out].set(bc)
    zb = jnp.zeros((1, h), jnp.float32)

    deg_call = _make_deg_kernel(n_pad, q0, q1, q_pad)
    scat_call = _make_scatter_kernel(n_pad, h, q0, q1, q_half)
    pre_call = _make_pre_kernel(n, n_pad, h)
    mid_call = _make_mid_kernel(n, n_pad, h, True)
    fin_call = _make_mid_kernel(n, n_pad, h, False)

    degp = deg_call(dst_p)
    p0 = degp[0].reshape(n_pad, 1)
    p1 = degp[1].reshape(n_pad, 1)
    dis, zt = pre_call(p0, p1, x_p, W1)

    for (w_next, b, g, be) in ((W2, b1, g1, be1), (W3, b2, g2, be2)):
        accp = scat_call(zt, src_p, dst_p)
        zt = mid_call(accp[0], accp[1], zt, dis,
                      b.reshape(1, h), g.reshape(1, h), be.reshape(1, h),
                      w_next, zb)

    accp = scat_call(zt, src_p, dst_p)
    outp = fin_call(accp[0], accp[1], zt, dis,
                    b3.reshape(1, h), g3.reshape(1, h), be3.reshape(1, h),
                    wc_p, bc_p)
    return outp[:n, :c_out]


# speedup vs baseline: 1.0139x; 1.0139x over previous
import jax
import jax.numpy as jnp
from jax import lax
from jax.experimental import pallas as pl
from jax.experimental.pallas import tpu as pltpu
from jax.experimental.pallas import tpu_sc as plsc

NC = 2
NS = 16
NW = NC * NS
L = 16
K = 128
SC0_FRAC = 0.62


def _sc_mesh():
    return plsc.VectorSubcoreMesh(core_axis_name="c", subcore_axis_name="s",
                                  num_cores=NC, num_subcores=NS)


def _make_deg_kernel(n_pad, q0, q1, q_max):
    rpt = n_pad // NS

    def body(dst_hbm, out_hbm, dst_v, ones_v, zero_v, deg_sp):
        c = lax.axis_index("c")
        s = lax.axis_index("s")
        wid = c * NS + s
        qw = jnp.where(c == 0, q0, q1)
        pltpu.sync_copy(dst_hbm.at[wid], dst_v)

        def init_ones(i, carry):
            ones_v[pl.ds(i * L, L)] = jnp.ones((L,), jnp.float32)
            return carry

        lax.fori_loop(0, K // L, init_ones, 0)

        def init_zero(i, carry):
            zero_v[pl.ds(i * L, L)] = jnp.zeros((L,), jnp.float32)
            return carry

        lax.fori_loop(0, rpt // L, init_zero, 0)
        pltpu.sync_copy(zero_v, deg_sp.at[pl.ds(s * rpt, rpt)])
        plsc.subcore_barrier()

        def chunk(j, carry):
            pltpu.sync_copy(ones_v, deg_sp.at[dst_v.at[j]], add=True)
            return carry

        lax.fori_loop(0, qw, chunk, 0)
        plsc.subcore_barrier()
        pltpu.sync_copy(deg_sp.at[pl.ds(s * rpt, rpt)],
                        out_hbm.at[c, pl.ds(s * rpt, rpt)])

    return pl.kernel(
        body,
        out_type=jax.ShapeDtypeStruct((NC, n_pad), jnp.float32),
        mesh=_sc_mesh(),
        scratch_types=[
            pltpu.VMEM((q_max, K), jnp.int32),
            pltpu.VMEM((K,), jnp.float32),
            pltpu.VMEM((rpt,), jnp.float32),
            pltpu.VMEM_SHARED((n_pad,), jnp.float32),
        ],
    )


def _make_scatter_kernel(n_pad, d, q0, q1, q_half):
    rpt = n_pad // NS

    def body(zt_hbm, src_hbm, dst_hbm, out_hbm, src_v, dst_v, rows0, rows1,
             acc_sp, sem0, sem1):
        c = lax.axis_index("c")
        s = lax.axis_index("s")
        wid = c * NS + s
        qw = jnp.where(c == 0, q0, q1)
        pltpu.sync_copy(zt_hbm.at[pl.ds(s * rpt, rpt)],
                        acc_sp.at[pl.ds(s * rpt, rpt)])
        plsc.subcore_barrier()

        def do_half(base, b):
            pltpu.sync_copy(src_hbm.at[wid, pl.ds(base, q_half)], src_v)
            pltpu.sync_copy(dst_hbm.at[wid, pl.ds(base, q_half)], dst_v)

            @pl.when(b > 0)
            def _():
                pltpu.async_copy(zt_hbm.at[src_v.at[0]], rows0, sem0)

            @pl.when(b > 1)
            def _():
                pltpu.async_copy(zt_hbm.at[src_v.at[1]], rows1, sem1)

            def chunk(j, carry):
                def step(rows, sem):
                    pltpu.make_async_copy(zt_hbm.at[src_v.at[j]], rows,
                                          sem).wait()
                    pltpu.sync_copy(rows, acc_sp.at[dst_v.at[j]], add=True)

                    @pl.when(j + 2 < b)
                    def _():
                        pltpu.async_copy(zt_hbm.at[src_v.at[j + 2]], rows,
                                         sem)

                @pl.when(j % 2 == 0)
                def _():
                    step(rows0, sem0)

                @pl.when(j % 2 == 1)
                def _():
                    step(rows1, sem1)

                return carry

            lax.fori_loop(0, b, chunk, 0)

        b0 = jnp.minimum(qw, q_half)
        do_half(0, b0)
        do_half(q_half, qw - b0)
        plsc.subcore_barrier()
        pltpu.sync_copy(acc_sp.at[pl.ds(s * rpt, rpt)],
                        out_hbm.at[c, pl.ds(s * rpt, rpt)])

    return pl.kernel(
        body,
        out_type=jax.ShapeDtypeStruct((NC, n_pad, d), jnp.float32),
        mesh=_sc_mesh(),
        scratch_types=[
            pltpu.VMEM((q_half, K), jnp.int32),
            pltpu.VMEM((q_half, K), jnp.int32),
            pltpu.VMEM((K, d), jnp.float32),
            pltpu.VMEM((K, d), jnp.float32),
            pltpu.VMEM_SHARED((n_pad, d), jnp.float32),
            pltpu.SemaphoreType.DMA,
            pltpu.SemaphoreType.DMA,
        ],
    )


def _make_pre_kernel(n, n_pad, h):
    def body(p0, p1, x, w, dis_out, zt_out):
        deg = p0[...] + p1[...] + 1.0
        rid = lax.broadcasted_iota(jnp.int32, (n_pad, 1), 0)
        dis = jnp.where(rid < n, lax.rsqrt(jnp.maximum(deg, 1e-12)), 0.0)
        dis_out[...] = dis
        zt_out[...] = jnp.dot(x[...], w[...],
                              preferred_element_type=jnp.float32) * dis

    return pl.pallas_call(
        body,
        out_shape=[
            jax.ShapeDtypeStruct((n_pad, 1), jnp.float32),
            jax.ShapeDtypeStruct((n_pad, h), jnp.float32),
        ],
    )


def _make_mid_kernel(n, n_pad, h, scale_out):
    def body(a0, a1, ztp, dis, b, g, be, w, bout, out):
        agg = a0[...] + a1[...] - ztp[...]
        y = agg * dis[...] + b[...]
        r = jnp.maximum(y, 0.0)
        mask = (lax.broadcasted_iota(jnp.int32, (n_pad, 1), 0)
                < n).astype(jnp.float32)
        mu = jnp.sum(r * mask, axis=0, keepdims=True) * (1.0 / n)
        dvn = (r - mu) * mask
        var = jnp.sum(dvn * dvn, axis=0, keepdims=True) * (1.0 / n)
        bn = (r - mu) * lax.rsqrt(var + 1e-5) * g[...] + be[...]
        res = jnp.dot(bn, w[...], preferred_element_type=jnp.float32)
        res = res + bout[...]
        if scale_out:
            res = res * dis[...]
        out[...] = res

    return pl.pallas_call(
        body,
        out_shape=jax.ShapeDtypeStruct((n_pad, h), jnp.float32),
    )


def kernel(x, edge_index, W1, b1, g1, be1, W2, b2, g2, be2, W3, b3, g3, be3,
           Wc, bc):
    n, d = x.shape
    h = W1.shape[1]
    e = edge_index.shape[1]
    c_out = Wc.shape[1]

    n_pad = ((n + 1 + 255) // 256) * 256
    nch = (e + NS * K - 1) // (NS * K)
    q0 = max(1, min(nch - 1, round(nch * SC0_FRAC)))
    q1 = nch - q0
    q_max = max(q0, q1)
    q_half = (((q_max + 1) // 2 + 7) // 8) * 8
    q_pad = 2 * q_half
    e_cov = NS * nch * K

    src = edge_index[0]
    dst = edge_index[1]
    pad_idx = jnp.full((e_cov - e,), n, jnp.int32)

    def slabs(idx):
        flat = jnp.concatenate([idx, pad_idx])
        c0 = flat[:NS * q0 * K].reshape(NS, q0, K)
        c1 = flat[NS * q0 * K:].reshape(NS, q1, K)
        pad0 = jnp.full((NS, q_pad - q0, K), n, jnp.int32)
        pad1 = jnp.full((NS, q_pad - q1, K), n, jnp.int32)
        return jnp.concatenate([jnp.concatenate([c0, pad0], axis=1),
                                jnp.concatenate([c1, pad1], axis=1)], axis=0)

    src_p = slabs(src)
    dst_p = slabs(dst)
    x_p = jnp.zeros((n_pad, d), jnp.float32).at[:n].set(x)
    wc_p = jnp.zeros((h, h), jnp.float32).at[:, :c_out].set(Wc)
    bc_p = jnp.zeros((1, h), jnp.float32).at[0, :c_out].set(bc)
    zb = jnp.zeros((1, h), jnp.float32)

    deg_call = _make_deg_kernel(n_pad, q0, q1, q_pad)
    scat_call = _make_scatter_kernel(n_pad, h, q0, q1, q_half)
    pre_call = _make_pre_kernel(n, n_pad, h)
    mid_call = _make_mid_kernel(n, n_pad, h, True)
    fin_call = _make_mid_kernel(n, n_pad, h, False)

    degp = deg_call(dst_p)
    p0 = degp[0].reshape(n_pad, 1)
    p1 = degp[1].reshape(n_pad, 1)
    dis, zt = pre_call(p0, p1, x_p, W1)

    for (w_next, b, g, be) in ((W2, b1, g1, be1), (W3, b2, g2, be2)):
        accp = scat_call(zt, src_p, dst_p)
        zt = mid_call(accp[0], accp[1], zt, dis,
                      b.reshape(1, h), g.reshape(1, h), be.reshape(1, h),
                      w_next, zb)

    accp = scat_call(zt, src_p, dst_p)
    outp = fin_call(accp[0], accp[1], zt, dis,
                    b3.reshape(1, h), g3.reshape(1, h), be3.reshape(1, h),
                    wc_p, bc_p)
    return outp[:n, :c_out]
